# baseline (speedup 1.0000x reference)
import functools

import jax
import jax.numpy as jnp
import numpy as np
from jax.experimental import pallas as pl
from jax.experimental.pallas import tpu as pltpu

N = 10000
E = 160000
F = 32
NB = 8
H = 64
NINT = 2
R_MAX = 5.0
AVG_NEIGH = 16.0

EBLK = 3200


def _edge_dense_body(rel_ref, rw0_ref, rw1_ref, rw2_ref,
                     sh_ref, w0_ref, w1_ref):
    inv_rmax = np.float32(1.0 / R_MAX)
    x = rel_ref[0:1, :] * inv_rmax
    y = rel_ref[1:2, :] * inv_rmax
    z = rel_ref[2:3, :] * inv_rmax
    r2 = x * x + y * y + z * z
    r = jnp.sqrt(r2)
    rinv = 1.0 / jnp.maximum(r, 1e-6)
    ux = x * rinv
    uy = y * rinv
    uz = z * rinv
    c1 = np.float32(np.sqrt(3.0))
    c2 = np.float32(np.sqrt(15.0))
    c3 = np.float32(np.sqrt(5.0) / 2.0)
    c4 = np.float32(np.sqrt(15.0) / 2.0)
    one = jnp.ones_like(ux)
    sh_rows = [
        one,
        c1 * ux, c1 * uy, c1 * uz,
        c2 * ux * uy, c2 * uy * uz, c3 * (3.0 * uz * uz - 1.0),
        c2 * ux * uz, c4 * (ux * ux - uy * uy),
    ]
    sh = jnp.concatenate(sh_rows, axis=0)
    sh_ref[0:9, :] = sh
    sh_ref[9:16, :] = jnp.zeros((7, EBLK), jnp.float32)

    rs = jnp.maximum(r, 1e-6)
    rs_inv = 1.0 / rs
    ns = (jax.lax.broadcasted_iota(jnp.int32, (NB, 1), 0) + 1).astype(jnp.float32)
    b = np.float32(np.sqrt(2.0)) * jnp.sin(ns * np.float32(np.pi) * rs) * rs_inv
    p = 6.0
    rp = rs ** 6
    env = (1.0 - (p + 1.0) * (p + 2.0) / 2.0 * rp
           + p * (p + 2.0) * rp * rs
           - p * (p + 1.0) / 2.0 * rp * rs * rs)
    env = jnp.where(rs < 1.0, env, 0.0)
    rb = b * env

    def mlp(i, out_ref):
        w = rw0_ref[i].T @ rb
        w = w * jax.nn.sigmoid(w)
        w = rw1_ref[i].T @ w
        w = w * jax.nn.sigmoid(w)
        out_ref[...] = rw2_ref[i].T @ w

    mlp(0, w0_ref)
    mlp(1, w1_ref)


def _edge_dense(rel_t, rw0, rw1, rw2):
    nblk = E // EBLK
    grid = (nblk,)
    return pl.pallas_call(
        _edge_dense_body,
        grid=grid,
        in_specs=[
            pl.BlockSpec((3, EBLK), lambda i: (0, i)),
            pl.BlockSpec((NINT, NB, H), lambda i: (0, 0, 0)),
            pl.BlockSpec((NINT, H, H), lambda i: (0, 0, 0)),
            pl.BlockSpec((NINT, H, F), lambda i: (0, 0, 0)),
        ],
        out_specs=[
            pl.BlockSpec((16, EBLK), lambda i: (0, i)),
            pl.BlockSpec((F, EBLK), lambda i: (0, i)),
            pl.BlockSpec((F, EBLK), lambda i: (0, i)),
        ],
        out_shape=[
            jax.ShapeDtypeStruct((16, E), jnp.float32),
            jax.ShapeDtypeStruct((F, E), jnp.float32),
            jax.ShapeDtypeStruct((F, E), jnp.float32),
        ],
    )(rel_t, rw0, rw1, rw2)


def kernel(positions, species, senders, receivers, embed_table, rw0, rw1, rw2, lin):
    rel = positions[receivers] - positions[senders]
    rel_t = rel.T
    sh_t, w0_t, w1_t = _edge_dense(rel_t, rw0, rw1, rw2)
    sh = sh_t[:9, :].T
    ws = (w0_t.T, w1_t.T)

    feats = jnp.zeros((N, F, 9), dtype=jnp.float32).at[:, :, 0].set(embed_table[species])
    for i in range(NINT):
        w = ws[i]
        fs = feats[senders]
        proj = jnp.einsum('efc,ec->ef', fs, sh)
        msg = jnp.einsum('ef,ec->efc', w * proj, sh)
        agg = jax.ops.segment_sum(msg, receivers, num_segments=N) / AVG_NEIGH
        new = jnp.concatenate([
            jnp.einsum('nfc,fg->ngc', agg[:, :, 0:1], lin[i, 0]),
            jnp.einsum('nfc,fg->ngc', agg[:, :, 1:4], lin[i, 1]),
            jnp.einsum('nfc,fg->ngc', agg[:, :, 4:9], lin[i, 2]),
        ], axis=2)
        if i > 0:
            new = new + feats
        feats = new
    ls = jnp.array([0, 1, 1, 1, 2, 2, 2, 2, 2], dtype=jnp.float32)
    alpha = 0.5 ** ls
    return feats * alpha[None, None, :]
